# baseline (speedup 1.0000x reference)
import functools

import jax
import jax.numpy as jnp
from jax import lax
from jax.experimental import pallas as pl
from jax.experimental.pallas import tpu as pltpu
from jax.experimental.pallas import tpu_sc as plsc

V = 100000
D = 128
BOS = 1
B = 4096
L = 200
N = B * L
NC, NS = 2, 16
NW = NC * NS
PER_W = N // NW
C = 128
CHUNKS_W = PER_W // C
NBUF = 5
NLANE = 16
NV = D // NLANE


def _sc_body(table_hbm, idx_hbm, out_hbm, idx_v, bos_v, *rest):
    bufs = rest[:NBUF]
    gsems = rest[NBUF:2 * NBUF]
    ssems = rest[2 * NBUF:3 * NBUF]
    wid = lax.axis_index("s") * NC + lax.axis_index("c")
    row0 = wid * PER_W

    pltpu.sync_copy(idx_hbm.at[pl.ds(row0, PER_W)], idx_v)
    pltpu.sync_copy(table_hbm.at[pl.ds(BOS, 1)], bos_v)

    lane = lax.iota(jnp.int32, NLANE)
    for m in range(PER_W // L):
        off = m * L
        sl = pl.ds((off // NLANE) * NLANE, NLANE)
        idx_v[sl] = jnp.where(lane == off % NLANE, BOS, idx_v[sl])

    bosv = [bos_v[0, pl.ds(v * NLANE, NLANE)] for v in range(NV)]

    NR = CHUNKS_W // NBUF

    def round_body(g, carry):
        j0 = g * NBUF
        gh = []
        for b in range(NBUF):
            @pl.when(g > 0)
            def _drain_prev_store(b=b):
                pltpu.make_async_copy(
                    bufs[b], out_hbm.at[pl.ds(row0, C)], ssems[b]).wait()

            gh.append(pltpu.async_copy(
                table_hbm.at[idx_v.at[pl.ds((j0 + b) * C, C)]],
                bufs[b], gsems[b]))
        for b in range(NBUF):
            gh[b].wait()
            buf = bufs[b]

            @plsc.parallel_loop(0, C, 1, unroll=2)
            def row_body(r):
                for v in range(NV):
                    sl = pl.ds(v * NLANE, NLANE)
                    buf[r, sl] = buf[r, sl] + bosv[v]

            pltpu.async_copy(
                buf, out_hbm.at[pl.ds(row0 + (j0 + b) * C, C)], ssems[b])
        return carry

    lax.fori_loop(0, NR, round_body, 0)
    for b in range(NBUF):
        pltpu.make_async_copy(
            bufs[b], out_hbm.at[pl.ds(row0, C)], ssems[b]).wait()


@functools.lru_cache(maxsize=1)
def _sc_kernel():
    mesh = plsc.VectorSubcoreMesh(
        core_axis_name="c", subcore_axis_name="s",
        num_cores=NC, num_subcores=NS)
    return pl.kernel(
        _sc_body,
        out_type=jax.ShapeDtypeStruct((N, D), jnp.float32),
        mesh=mesh,
        scratch_types=[
            pltpu.VMEM((PER_W,), jnp.int32),
            pltpu.VMEM((1, D), jnp.float32),
            *[pltpu.VMEM((C, D), jnp.float32) for _ in range(NBUF)],
            *[pltpu.SemaphoreType.DMA for _ in range(2 * NBUF)],
        ],
    )


def kernel(tokens, table):
    idx = tokens.astype(jnp.int32).reshape(N)
    out = _sc_kernel()(table, idx)
    return out.reshape(B, L, D)
